# baseline (speedup 1.0000x reference)
import functools

import jax
import jax.numpy as jnp
from jax import lax
from jax.experimental import pallas as pl
from jax.experimental.pallas import tpu as pltpu
from jax.experimental.pallas import tpu_sc as plsc

_VOCAB = 1000000
_D = 64
_B = 16384
_K = 10
_S = 50
_DELTA = 1.0
_GAMMA = 1.0

_NC = 2
_NS = 16
_NW = _NC * _NS
_BPW = _B // _NW
_CB = 8
_NCH = _BPW // _CB
_HB = 4
_NH = _BPW // _HB
_LN = 16


def _sc_body(cen_hbm, true_hbm, neg_hbm, seq_hbm, emb_hbm,
             ct_hbm, negt_hbm, seqsum_hbm,
             cen_idx, true_idx, neg_idx, seq_idx,
             cen0, cen1, true0, true1, neg0, neg1, seq0, seq1, so0, so1,
             ct_out, negt_out, g0, g1, s0, s1, osem):
  wid = lax.axis_index("s") * _NC + lax.axis_index("c")
  base = wid * _BPW

  pltpu.sync_copy(cen_hbm.at[pl.ds(base, _BPW)], cen_idx)
  pltpu.sync_copy(true_hbm.at[pl.ds(base, _BPW)], true_idx)
  pltpu.sync_copy(neg_hbm.at[pl.ds(base * _K, _BPW * _K)], neg_idx)
  pltpu.sync_copy(seq_hbm.at[pl.ds(base * _S, _BPW * _S)], seq_idx)

  gsems = (g0, g1)
  ssems = (s0, s1)
  cens = (cen0, cen1)
  trues = (true0, true1)
  negs = (neg0, neg1)
  seqs = (seq0, seq1)
  souts = (so0, so1)

  def dot_copies(par, c):
    return [
        pltpu.make_async_copy(
            emb_hbm.at[cen_idx.at[pl.ds(c * _CB, _CB)]], cens[par], gsems[par]),
        pltpu.make_async_copy(
            emb_hbm.at[true_idx.at[pl.ds(c * _CB, _CB)]], trues[par],
            gsems[par]),
        pltpu.make_async_copy(
            emb_hbm.at[neg_idx.at[pl.ds(c * _CB * _K, _CB * _K)]], negs[par],
            gsems[par]),
    ]

  def seq_copies(hp, h):
    return [
        pltpu.make_async_copy(
            emb_hbm.at[seq_idx.at[pl.ds(h * _HB * _S, 104)]],
            seqs[hp].at[pl.ds(0, 104)], ssems[hp]),
        pltpu.make_async_copy(
            emb_hbm.at[seq_idx.at[pl.ds(h * _HB * _S + 104, 96)]],
            seqs[hp].at[pl.ds(104, 96)], ssems[hp]),
    ]

  def fire_dots(par, c):
    for cp in dot_copies(par, c):
      cp.start()

  def drain_dots(par, c):
    for cp in dot_copies(par, c):
      cp.wait()

  def fire_seq(hp, h):
    for cp in seq_copies(hp, h):
      cp.start()

  def drain_seq(hp, h):
    for cp in seq_copies(hp, h):
      cp.wait()

  fire_dots(0, 0)
  fire_dots(1, 1)
  fire_seq(0, 0)
  fire_seq(1, 1)

  lanes = lax.iota(jnp.int32, _LN)

  @pl.loop(0, _NCH, step=2)
  def _chunk_pair(g):
    ct_vec = jnp.zeros((_LN,), jnp.float32)
    neg_vecs = [jnp.zeros((_LN,), jnp.float32) for _ in range(_K)]
    for par in range(2):
      c = g + par
      drain_dots(par, c)

      for r in range(_CB):
        ln = par * _CB + r
        cv = [cens[par][r, pl.ds(16 * j, 16)] for j in range(4)]
        tv = [trues[par][r, pl.ds(16 * j, 16)] for j in range(4)]
        p = (cv[0] * tv[0] + cv[1] * tv[1]) + (cv[2] * tv[2] + cv[3] * tv[3])
        ct_vec = jnp.where(lanes == ln, jnp.sum(p), ct_vec)
        for k in range(_K):
          nv = [negs[par][r * _K + k, pl.ds(16 * j, 16)] for j in range(4)]
          q = (cv[0] * nv[0] + cv[1] * nv[1]) + (cv[2] * nv[2] + cv[3] * nv[3])
          neg_vecs[k] = jnp.where(lanes == ln, jnp.sum(q), neg_vecs[k])

      @pl.when(c + 2 < _NCH)
      def _():
        fire_dots(par, c + 2)

      for hp in range(2):
        h = 2 * c + hp
        drain_seq(hp, h)

        @pl.when(h >= 2)
        def _():
          pltpu.make_async_copy(
              souts[hp], seqsum_hbm.at[pl.ds(0, _HB * _D)], osem).wait()

        for r4 in range(_HB):
          rowbase = r4 * _S
          for j in range(4):
            col = pl.ds(16 * j, 16)
            a0 = seqs[hp][rowbase, col]
            a1 = seqs[hp][rowbase + 1, col]
            for si in range(2, _S, 2):
              a0 = a0 + seqs[hp][rowbase + si, col]
              a1 = a1 + seqs[hp][rowbase + si + 1, col]
            souts[hp][pl.ds(r4 * _D + 16 * j, 16)] = a0 + a1

        pltpu.make_async_copy(
            souts[hp],
            seqsum_hbm.at[pl.ds((base + h * _HB) * _D, _HB * _D)],
            osem).start()

        @pl.when(h + 2 < _NH)
        def _():
          fire_seq(hp, h + 2)

    ct_out[pl.ds(g * _CB, 16)] = ct_vec
    for k in range(_K):
      negt_out[k, pl.ds(g * _CB, 16)] = neg_vecs[k]

  for hp in range(2):
    pltpu.make_async_copy(
        souts[hp], seqsum_hbm.at[pl.ds(0, _HB * _D)], osem).wait()

  pltpu.sync_copy(ct_out, ct_hbm.at[pl.ds(base, _BPW)])
  for k in range(_K):
    pltpu.sync_copy(negt_out.at[k], negt_hbm.at[pl.ds(k * _B + base, _BPW)])


_sc_lookup = functools.partial(
    pl.kernel,
    out_type=(
        jax.ShapeDtypeStruct((_B,), jnp.float32),
        jax.ShapeDtypeStruct((_K * _B,), jnp.float32),
        jax.ShapeDtypeStruct((_B * _D,), jnp.float32),
    ),
    mesh=plsc.VectorSubcoreMesh(
        core_axis_name="c", subcore_axis_name="s", num_cores=_NC),
    compiler_params=pltpu.CompilerParams(
        needs_layout_passes=False, use_tc_tiling_on_sc=False),
    scratch_types=[
        pltpu.VMEM((_BPW,), jnp.int32),
        pltpu.VMEM((_BPW,), jnp.int32),
        pltpu.VMEM((_BPW * _K,), jnp.int32),
        pltpu.VMEM((_BPW * _S,), jnp.int32),
        pltpu.VMEM((_CB, _D), jnp.float32),
        pltpu.VMEM((_CB, _D), jnp.float32),
        pltpu.VMEM((_CB, _D), jnp.float32),
        pltpu.VMEM((_CB, _D), jnp.float32),
        pltpu.VMEM((_CB * _K, _D), jnp.float32),
        pltpu.VMEM((_CB * _K, _D), jnp.float32),
        pltpu.VMEM((_HB * _S, _D), jnp.float32),
        pltpu.VMEM((_HB * _S, _D), jnp.float32),
        pltpu.VMEM((_HB * _D,), jnp.float32),
        pltpu.VMEM((_HB * _D,), jnp.float32),
        pltpu.VMEM((_BPW,), jnp.float32),
        pltpu.VMEM((_K, _BPW), jnp.float32),
        pltpu.SemaphoreType.DMA,
        pltpu.SemaphoreType.DMA,
        pltpu.SemaphoreType.DMA,
        pltpu.SemaphoreType.DMA,
        pltpu.SemaphoreType.DMA,
    ],
)(_sc_body)


def _tc_body(ct_ref, negt_ref, seqsum_ref, w_ref, b_ref, lab_ref, out_ref):
  def logsig(x):
    return jnp.minimum(x, 0.0) - jnp.log1p(jnp.exp(-jnp.abs(x)))

  obj_sum = jnp.sum(logsig(ct_ref[...]))
  neg_sum = jnp.sum(logsig(-negt_ref[...]))
  deno_loss = -(obj_sum + neg_sum) / _B

  seq_repr = seqsum_ref[...] * (1.0 / _S)
  w = w_ref[...]
  b = b_ref[...]
  l0 = jnp.sum(seq_repr * w[:, 0][None, :], axis=1, keepdims=True) + b[0, 0]
  l1 = jnp.sum(seq_repr * w[:, 1][None, :], axis=1, keepdims=True) + b[0, 1]
  m = jnp.maximum(l0, l1)
  z = m + jnp.log(jnp.exp(l0 - m) + jnp.exp(l1 - m))
  picked = jnp.where(lab_ref[...] == 0, l0, l1) - z
  cono_loss = -jnp.sum(picked) / _B

  def sigmoid(x):
    return 1.0 / (1.0 + jnp.exp(-x))

  dec_loss = 1.0 + _DELTA * sigmoid(deno_loss) + _GAMMA * sigmoid(cono_loss)
  idx = lax.broadcasted_iota(jnp.int32, (1, 4), 1)
  out_ref[...] = jnp.where(
      idx == 0, dec_loss, jnp.where(idx == 1, deno_loss, cono_loss))


def kernel(center_word_ids, true_context_ids, negative_context_ids,
           seq_word_ids, cono_labels, embedding, W_cono, b_cono):
  cen = center_word_ids.astype(jnp.int32)
  tru = true_context_ids.astype(jnp.int32)
  neg = negative_context_ids.astype(jnp.int32).reshape(_B * _K)
  seq = seq_word_ids.astype(jnp.int32).reshape(_B * _S)

  ct, negt, seqsum = _sc_lookup(cen, tru, neg, seq, embedding)

  out = pl.pallas_call(
      _tc_body,
      out_shape=jax.ShapeDtypeStruct((1, 4), jnp.float32),
  )(ct.reshape(128, 128), negt.reshape(_K, _B), seqsum.reshape(_B, _D),
    W_cono, b_cono.reshape(1, 2),
    cono_labels.astype(jnp.int32).reshape(_B, 1))

  v = out.reshape(4)
  return (v[0], v[1], v[2], v[3])


# speedup vs baseline: 1.0446x; 1.0446x over previous
import functools

import jax
import jax.numpy as jnp
from jax import lax
from jax.experimental import pallas as pl
from jax.experimental.pallas import tpu as pltpu
from jax.experimental.pallas import tpu_sc as plsc

_VOCAB = 1000000
_D = 64
_B = 16384
_K = 10
_S = 50
_DELTA = 1.0
_GAMMA = 1.0

_NC = 1
_NS = 16
_NW = _NC * _NS
_BPW = _B // _NW
_CB = 8
_NCH = _BPW // _CB
_HB = 4
_NH = _BPW // _HB
_LN = 16
_DR = _K + 1


def _sc_body(cen_hbm, true_hbm, neg_hbm, seq_hbm, emb_hbm,
             dots_hbm, seqsum_hbm,
             cen_idx, true_idx, neg_idx, seq_idx,
             cen0, cen1, true0, true1, neg0, neg1, seq0, seq1, so0, so1,
             ob0, ob1, g0, g1, s0, s1, osem, dsem):
  wid = lax.axis_index("s") * _NC + lax.axis_index("c")
  base = wid * _BPW

  pltpu.sync_copy(cen_hbm.at[pl.ds(base, _BPW)], cen_idx)
  pltpu.sync_copy(true_hbm.at[pl.ds(base, _BPW)], true_idx)
  pltpu.sync_copy(neg_hbm.at[pl.ds(base * _K, _BPW * _K)], neg_idx)
  pltpu.sync_copy(seq_hbm.at[pl.ds(base * _S, _BPW * _S)], seq_idx)

  gsems = (g0, g1)
  ssems = (s0, s1)
  cens = (cen0, cen1)
  trues = (true0, true1)
  negs = (neg0, neg1)
  seqs = (seq0, seq1)
  souts = (so0, so1)
  obufs = (ob0, ob1)

  def dot_copies(par, c):
    return [
        pltpu.make_async_copy(
            emb_hbm.at[cen_idx.at[pl.ds(c * _CB, _CB)]], cens[par], gsems[par]),
        pltpu.make_async_copy(
            emb_hbm.at[true_idx.at[pl.ds(c * _CB, _CB)]], trues[par],
            gsems[par]),
        pltpu.make_async_copy(
            emb_hbm.at[neg_idx.at[pl.ds(c * _CB * _K, _CB * _K)]], negs[par],
            gsems[par]),
    ]

  def seq_copies(hp, h):
    return [
        pltpu.make_async_copy(
            emb_hbm.at[seq_idx.at[pl.ds(h * _HB * _S, 104)]],
            seqs[hp].at[pl.ds(0, 104)], ssems[hp]),
        pltpu.make_async_copy(
            emb_hbm.at[seq_idx.at[pl.ds(h * _HB * _S + 104, 96)]],
            seqs[hp].at[pl.ds(104, 96)], ssems[hp]),
    ]

  def fire_dots(par, c):
    for cp in dot_copies(par, c):
      cp.start()

  def drain_dots(par, c):
    for cp in dot_copies(par, c):
      cp.wait()

  def fire_seq(hp, h):
    for cp in seq_copies(hp, h):
      cp.start()

  def drain_seq(hp, h):
    for cp in seq_copies(hp, h):
      cp.wait()

  fire_dots(0, 0)
  fire_dots(1, 1)
  fire_seq(0, 0)
  fire_seq(1, 1)

  lanes = lax.iota(jnp.int32, _LN)

  @pl.loop(0, _NCH, step=2)
  def _chunk_pair(g):
    ct_vec = jnp.zeros((_LN,), jnp.float32)
    neg_vecs = [jnp.zeros((_LN,), jnp.float32) for _ in range(_K)]
    for par in range(2):
      c = g + par
      drain_dots(par, c)

      for r in range(_CB):
        ln = par * _CB + r
        cv = [cens[par][r, pl.ds(16 * j, 16)] for j in range(4)]
        tv = [trues[par][r, pl.ds(16 * j, 16)] for j in range(4)]
        p = (cv[0] * tv[0] + cv[1] * tv[1]) + (cv[2] * tv[2] + cv[3] * tv[3])
        ct_vec = jnp.where(lanes == ln, jnp.sum(p), ct_vec)
        for k in range(_K):
          nv = [negs[par][r * _K + k, pl.ds(16 * j, 16)] for j in range(4)]
          q = (cv[0] * nv[0] + cv[1] * nv[1]) + (cv[2] * nv[2] + cv[3] * nv[3])
          neg_vecs[k] = jnp.where(lanes == ln, jnp.sum(q), neg_vecs[k])

      @pl.when(c + 2 < _NCH)
      def _():
        fire_dots(par, c + 2)

      for hp in range(2):
        h = 2 * c + hp
        drain_seq(hp, h)

        @pl.when(h >= 2)
        def _():
          pltpu.make_async_copy(
              souts[hp], seqsum_hbm.at[pl.ds(0, _HB * _D)], osem).wait()

        for r4 in range(_HB):
          rowbase = r4 * _S
          init = tuple(jnp.zeros((_LN,), jnp.float32) for _ in range(4))

          @pl.loop(0, _S, init_carry=init, step=10)
          def acc(si, carry):
            out = carry
            for u in range(10):
              rid = jnp.full((_LN,), rowbase + si + u, jnp.int32)
              out = tuple(
                  a + plsc.load_gather(seqs[hp], [rid, lanes + 16 * j])
                  for j, a in enumerate(out))
            return out

          for j in range(4):
            souts[hp][pl.ds(r4 * _D + 16 * j, 16)] = acc[j]

        pltpu.make_async_copy(
            souts[hp],
            seqsum_hbm.at[pl.ds((base + h * _HB) * _D, _HB * _D)],
            osem).start()

        @pl.when(h + 2 < _NH)
        def _():
          fire_seq(hp, h + 2)

    pp = (g // 2) % 2
    for pq in range(2):
      @pl.when((g >= 4) & (pp == pq))
      def _():
        pltpu.make_async_copy(
            obufs[pq], dots_hbm.at[pl.ds(0, _DR * _LN)], dsem).wait()
      @pl.when(pp == pq)
      def _():
        obufs[pq][pl.ds(0, 16)] = ct_vec
        for k in range(_K):
          obufs[pq][pl.ds((k + 1) * _LN, 16)] = neg_vecs[k]
        pltpu.make_async_copy(
            obufs[pq],
            dots_hbm.at[pl.ds((wid * (_NCH // 2) + g // 2) * _DR * _LN,
                              _DR * _LN)],
            dsem).start()

  for hp in range(2):
    pltpu.make_async_copy(
        souts[hp], seqsum_hbm.at[pl.ds(0, _HB * _D)], osem).wait()
  for pq in range(2):
    pltpu.make_async_copy(
        obufs[pq], dots_hbm.at[pl.ds(0, _DR * _LN)], dsem).wait()


_sc_lookup = functools.partial(
    pl.kernel,
    out_type=(
        jax.ShapeDtypeStruct((_B // _LN * _DR * _LN,), jnp.float32),
        jax.ShapeDtypeStruct((_B * _D,), jnp.float32),
    ),
    mesh=plsc.VectorSubcoreMesh(
        core_axis_name="c", subcore_axis_name="s", num_cores=_NC),
    compiler_params=pltpu.CompilerParams(
        needs_layout_passes=False, use_tc_tiling_on_sc=False),
    scratch_types=[
        pltpu.VMEM((_BPW,), jnp.int32),
        pltpu.VMEM((_BPW,), jnp.int32),
        pltpu.VMEM((_BPW * _K,), jnp.int32),
        pltpu.VMEM((_BPW * _S,), jnp.int32),
        pltpu.VMEM((_CB, _D), jnp.float32),
        pltpu.VMEM((_CB, _D), jnp.float32),
        pltpu.VMEM((_CB, _D), jnp.float32),
        pltpu.VMEM((_CB, _D), jnp.float32),
        pltpu.VMEM((_CB * _K, _D), jnp.float32),
        pltpu.VMEM((_CB * _K, _D), jnp.float32),
        pltpu.VMEM((_HB * _S, _D), jnp.float32),
        pltpu.VMEM((_HB * _S, _D), jnp.float32),
        pltpu.VMEM((_HB * _D,), jnp.float32),
        pltpu.VMEM((_HB * _D,), jnp.float32),
        pltpu.VMEM((_DR * _LN,), jnp.float32),
        pltpu.VMEM((_DR * _LN,), jnp.float32),
        pltpu.SemaphoreType.DMA,
        pltpu.SemaphoreType.DMA,
        pltpu.SemaphoreType.DMA,
        pltpu.SemaphoreType.DMA,
        pltpu.SemaphoreType.DMA,
        pltpu.SemaphoreType.DMA,
    ],
)(_sc_body)


def _tc_body(dots_ref, seqsum_ref, w_ref, b_ref, lab_ref, out_ref):
  def logsig(x):
    return jnp.minimum(x, 0.0) - jnp.log1p(jnp.exp(-jnp.abs(x)))

  d = dots_ref[...]
  obj_sum = jnp.sum(logsig(d[:, 0:_LN]))
  neg_sum = jnp.sum(logsig(-d[:, _LN:]))
  deno_loss = -(obj_sum + neg_sum) / _B

  seq_repr = seqsum_ref[...] * (1.0 / _S)
  w = w_ref[...]
  b = b_ref[...]
  l0 = jnp.sum(seq_repr * w[:, 0][None, :], axis=1, keepdims=True) + b[0, 0]
  l1 = jnp.sum(seq_repr * w[:, 1][None, :], axis=1, keepdims=True) + b[0, 1]
  m = jnp.maximum(l0, l1)
  z = m + jnp.log(jnp.exp(l0 - m) + jnp.exp(l1 - m))
  picked = jnp.where(lab_ref[...] == 0, l0, l1) - z
  cono_loss = -jnp.sum(picked) / _B

  def sigmoid(x):
    return 1.0 / (1.0 + jnp.exp(-x))

  dec_loss = 1.0 + _DELTA * sigmoid(deno_loss) + _GAMMA * sigmoid(cono_loss)
  idx = lax.broadcasted_iota(jnp.int32, (1, 4), 1)
  out_ref[...] = jnp.where(
      idx == 0, dec_loss, jnp.where(idx == 1, deno_loss, cono_loss))


def kernel(center_word_ids, true_context_ids, negative_context_ids,
           seq_word_ids, cono_labels, embedding, W_cono, b_cono):
  cen = center_word_ids.astype(jnp.int32)
  tru = true_context_ids.astype(jnp.int32)
  neg = negative_context_ids.astype(jnp.int32).reshape(_B * _K)
  seq = seq_word_ids.astype(jnp.int32).reshape(_B * _S)

  dots, seqsum = _sc_lookup(cen, tru, neg, seq, embedding)

  out = pl.pallas_call(
      _tc_body,
      out_shape=jax.ShapeDtypeStruct((1, 4), jnp.float32),
  )(dots.reshape(_B // _LN, _DR * _LN), seqsum.reshape(_B, _D),
    W_cono, b_cono.reshape(1, 2),
    cono_labels.astype(jnp.int32).reshape(_B, 1))

  v = out.reshape(4)
  return (v[0], v[1], v[2], v[3])


# speedup vs baseline: 1.2754x; 1.2209x over previous
import functools

import jax
import jax.numpy as jnp
from jax import lax
from jax.experimental import pallas as pl
from jax.experimental.pallas import tpu as pltpu
from jax.experimental.pallas import tpu_sc as plsc

_VOCAB = 1000000
_D = 64
_DP = 128
_B = 16384
_K = 10
_S = 50
_DELTA = 1.0
_GAMMA = 1.0

_NC = 2
_NS = 16
_NW = _NC * _NS
_BPW = _B // _NW
_CB = 8
_NCH = _BPW // _CB
_HB = 4
_NH = _BPW // _HB
_LN = 16


def _sc_body(cen_hbm, true_hbm, neg_hbm, seq_hbm, emb_hbm,
             ct_hbm, negt_hbm, seqsum_hbm,
             cen_idx, true_idx, neg_idx, seq_idx,
             cen0, cen1, true0, true1, neg0, neg1, seq0, seq1, so0, so1,
             ct_out, negt_out, g0, g1, s0, s1, osem):
  wid = lax.axis_index("s") * _NC + lax.axis_index("c")
  base = wid * _BPW

  pltpu.sync_copy(cen_hbm.at[pl.ds(base, _BPW)], cen_idx)
  pltpu.sync_copy(true_hbm.at[pl.ds(base, _BPW)], true_idx)
  pltpu.sync_copy(neg_hbm.at[pl.ds(base * _K, _BPW * _K)], neg_idx)
  pltpu.sync_copy(seq_hbm.at[pl.ds(base * _S, _BPW * _S)], seq_idx)

  gsems = (g0, g1)
  ssems = (s0, s1)
  cens = (cen0, cen1)
  trues = (true0, true1)
  negs = (neg0, neg1)
  seqs = (seq0, seq1)
  souts = (so0, so1)

  def dot_copies(par, c):
    return [
        pltpu.make_async_copy(
            emb_hbm.at[cen_idx.at[pl.ds(c * _CB, _CB)]], cens[par], gsems[par]),
        pltpu.make_async_copy(
            emb_hbm.at[true_idx.at[pl.ds(c * _CB, _CB)]], trues[par],
            gsems[par]),
        pltpu.make_async_copy(
            emb_hbm.at[neg_idx.at[pl.ds(c * _CB * _K, _CB * _K)]], negs[par],
            gsems[par]),
    ]

  def seq_copies(hp, h):
    return [
        pltpu.make_async_copy(
            emb_hbm.at[seq_idx.at[pl.ds(h * _HB * _S, 104)]],
            seqs[hp].at[pl.ds(0, 104)], ssems[hp]),
        pltpu.make_async_copy(
            emb_hbm.at[seq_idx.at[pl.ds(h * _HB * _S + 104, 96)]],
            seqs[hp].at[pl.ds(104, 96)], ssems[hp]),
    ]

  def fire_dots(par, c):
    for cp in dot_copies(par, c):
      cp.start()

  def drain_dots(par, c):
    for cp in dot_copies(par, c):
      cp.wait()

  def fire_seq(hp, h):
    for cp in seq_copies(hp, h):
      cp.start()

  def drain_seq(hp, h):
    for cp in seq_copies(hp, h):
      cp.wait()

  fire_dots(0, 0)
  fire_dots(1, 1)
  fire_seq(0, 0)
  fire_seq(1, 1)

  lanes = lax.iota(jnp.int32, _LN)

  @pl.loop(0, _NCH, step=2)
  def _chunk_pair(g):
    ct_vec = jnp.zeros((_LN,), jnp.float32)
    neg_vecs = [jnp.zeros((_LN,), jnp.float32) for _ in range(_K)]
    for par in range(2):
      c = g + par
      drain_dots(par, c)

      for r in range(_CB):
        ln = par * _CB + r
        cv = [cens[par][r, pl.ds(16 * j, 16)] for j in range(4)]
        tv = [trues[par][r, pl.ds(16 * j, 16)] for j in range(4)]
        p = (cv[0] * tv[0] + cv[1] * tv[1]) + (cv[2] * tv[2] + cv[3] * tv[3])
        ct_vec = jnp.where(lanes == ln, jnp.sum(p), ct_vec)
        for k in range(_K):
          nv = [negs[par][r * _K + k, pl.ds(16 * j, 16)] for j in range(4)]
          q = (cv[0] * nv[0] + cv[1] * nv[1]) + (cv[2] * nv[2] + cv[3] * nv[3])
          neg_vecs[k] = jnp.where(lanes == ln, jnp.sum(q), neg_vecs[k])

      @pl.when(c + 2 < _NCH)
      def _():
        fire_dots(par, c + 2)

      for hp in range(2):
        h = 2 * c + hp
        drain_seq(hp, h)

        @pl.when(h >= 2)
        def _():
          pltpu.make_async_copy(
              souts[hp], seqsum_hbm.at[pl.ds(0, _HB * _D)], osem).wait()

        for r4 in range(_HB):
          rowbase = r4 * _S
          init = tuple(jnp.zeros((_LN,), jnp.float32) for _ in range(4))

          @pl.loop(0, _S, init_carry=init, step=10)
          def acc(si, carry):
            out = carry
            for u in range(10):
              rid = jnp.full((_LN,), rowbase + si + u, jnp.int32)
              out = tuple(
                  a + plsc.load_gather(seqs[hp], [rid, lanes + 16 * j])
                  for j, a in enumerate(out))
            return out

          for j in range(4):
            souts[hp][pl.ds(r4 * _D + 16 * j, 16)] = acc[j]

        pltpu.make_async_copy(
            souts[hp],
            seqsum_hbm.at[pl.ds((base + h * _HB) * _D, _HB * _D)],
            osem).start()

        @pl.when(h + 2 < _NH)
        def _():
          fire_seq(hp, h + 2)

    ct_out[pl.ds(g * _CB, 16)] = ct_vec
    for k in range(_K):
      negt_out[k, pl.ds(g * _CB, 16)] = neg_vecs[k]

  for hp in range(2):
    pltpu.make_async_copy(
        souts[hp], seqsum_hbm.at[pl.ds(0, _HB * _D)], osem).wait()

  pltpu.sync_copy(ct_out, ct_hbm.at[pl.ds(base, _BPW)])
  for k in range(_K):
    pltpu.sync_copy(negt_out.at[k], negt_hbm.at[pl.ds(k * _B + base, _BPW)])


_sc_lookup = functools.partial(
    pl.kernel,
    out_type=(
        jax.ShapeDtypeStruct((_B,), jnp.float32),
        jax.ShapeDtypeStruct((_K * _B,), jnp.float32),
        jax.ShapeDtypeStruct((_B * _D,), jnp.float32),
    ),
    mesh=plsc.VectorSubcoreMesh(
        core_axis_name="c", subcore_axis_name="s", num_cores=_NC),
    compiler_params=pltpu.CompilerParams(
        needs_layout_passes=False, use_tc_tiling_on_sc=True),
    scratch_types=[
        pltpu.VMEM((_BPW,), jnp.int32),
        pltpu.VMEM((_BPW,), jnp.int32),
        pltpu.VMEM((_BPW * _K,), jnp.int32),
        pltpu.VMEM((_BPW * _S,), jnp.int32),
        pltpu.VMEM((_CB, _DP), jnp.float32),
        pltpu.VMEM((_CB, _DP), jnp.float32),
        pltpu.VMEM((_CB, _DP), jnp.float32),
        pltpu.VMEM((_CB, _DP), jnp.float32),
        pltpu.VMEM((_CB * _K, _DP), jnp.float32),
        pltpu.VMEM((_CB * _K, _DP), jnp.float32),
        pltpu.VMEM((_HB * _S, _DP), jnp.float32),
        pltpu.VMEM((_HB * _S, _DP), jnp.float32),
        pltpu.VMEM((_HB * _D,), jnp.float32),
        pltpu.VMEM((_HB * _D,), jnp.float32),
        pltpu.VMEM((_BPW,), jnp.float32),
        pltpu.VMEM((_K, _BPW), jnp.float32),
        pltpu.SemaphoreType.DMA,
        pltpu.SemaphoreType.DMA,
        pltpu.SemaphoreType.DMA,
        pltpu.SemaphoreType.DMA,
        pltpu.SemaphoreType.DMA,
    ],
)(_sc_body)


def _tc_body(ct_ref, negt_ref, seqsum_ref, w_ref, b_ref, lab_ref, out_ref):
  def logsig(x):
    return jnp.minimum(x, 0.0) - jnp.log1p(jnp.exp(-jnp.abs(x)))

  obj_sum = jnp.sum(logsig(ct_ref[...]))
  neg_sum = jnp.sum(logsig(-negt_ref[...]))
  deno_loss = -(obj_sum + neg_sum) / _B

  seq_repr = seqsum_ref[...] * (1.0 / _S)
  w = w_ref[...]
  b = b_ref[...]
  l0 = jnp.sum(seq_repr * w[:, 0][None, :], axis=1, keepdims=True) + b[0, 0]
  l1 = jnp.sum(seq_repr * w[:, 1][None, :], axis=1, keepdims=True) + b[0, 1]
  m = jnp.maximum(l0, l1)
  z = m + jnp.log(jnp.exp(l0 - m) + jnp.exp(l1 - m))
  picked = jnp.where(lab_ref[...] == 0, l0, l1) - z
  cono_loss = -jnp.sum(picked) / _B

  def sigmoid(x):
    return 1.0 / (1.0 + jnp.exp(-x))

  dec_loss = 1.0 + _DELTA * sigmoid(deno_loss) + _GAMMA * sigmoid(cono_loss)
  idx = lax.broadcasted_iota(jnp.int32, (1, 4), 1)
  out_ref[...] = jnp.where(
      idx == 0, dec_loss, jnp.where(idx == 1, deno_loss, cono_loss))


def kernel(center_word_ids, true_context_ids, negative_context_ids,
           seq_word_ids, cono_labels, embedding, W_cono, b_cono):
  cen = center_word_ids.astype(jnp.int32)
  tru = true_context_ids.astype(jnp.int32)
  neg = negative_context_ids.astype(jnp.int32).reshape(_B * _K)
  seq = seq_word_ids.astype(jnp.int32).reshape(_B * _S)

  emb128 = jnp.pad(embedding, ((0, 0), (0, _DP - _D)))
  ct, negt, seqsum = _sc_lookup(cen, tru, neg, seq, emb128)

  out = pl.pallas_call(
      _tc_body,
      out_shape=jax.ShapeDtypeStruct((1, 4), jnp.float32),
  )(ct.reshape(128, 128), negt.reshape(_K, _B), seqsum.reshape(_B, _D),
    W_cono, b_cono.reshape(1, 2),
    cono_labels.astype(jnp.int32).reshape(_B, 1))

  v = out.reshape(4)
  return (v[0], v[1], v[2], v[3])
